# baseline (speedup 1.0000x reference)
import functools

import jax
import jax.numpy as jnp
from jax import lax
from jax.experimental import pallas as pl
from jax.experimental.pallas import tpu as pltpu
from jax.experimental.pallas import tpu_sc as plsc

F = 26
B = 1024
L = 200
CARD1 = 100001
PITCH = 100032
PCARD1 = 1001
PROD_OFF = F * PITCH
TROWS = PROD_OFF + 1024
EMB = 16
NSLOT = F + 1
SLAB = L * NSLOT
M = B * SLAB
NW = 32
BPW = B // NW
NS = SLAB // 128
TAIL = SLAB - NS * 128

CHROWS = 1024
CPF = 98
CTAIL = CARD1 - (CPF - 1) * CHROWS
NCHK = F * CPF
RSTEPS = -(-NCHK // NW)


def _flat_kernel(tab3_hbm, ptab_hbm, out1_hbm, buf_v, psem):
    c = lax.axis_index("c")
    s = lax.axis_index("s")
    wid = s * 2 + c

    def step(g, carry):
        ck = g * NW + wid

        @pl.when(ck < NCHK)
        def _():
            f = ck // CPF
            j = ck % CPF
            r0 = j * CHROWS

            @pl.when(j < CPF - 1)
            def _():
                pltpu.sync_copy(tab3_hbm.at[f, pl.ds(r0, CHROWS), :], buf_v)
                pltpu.sync_copy(
                    buf_v, out1_hbm.at[pl.ds(f * PITCH + r0, CHROWS), :])

            @pl.when(j == CPF - 1)
            def _():
                pltpu.sync_copy(
                    tab3_hbm.at[f, pl.ds(r0, CTAIL), :],
                    buf_v.at[pl.ds(0, CTAIL), :])
                pltpu.sync_copy(
                    buf_v.at[pl.ds(0, CTAIL), :],
                    out1_hbm.at[pl.ds(f * PITCH + r0, CTAIL), :])

        return carry

    lax.fori_loop(0, RSTEPS, step, 0)

    @pl.when(wid == 0)
    def _():
        pltpu.sync_copy(ptab_hbm, buf_v.at[pl.ds(0, PCARD1), :])
        pltpu.sync_copy(
            buf_v.at[pl.ds(0, PCARD1), :],
            out1_hbm.at[pl.ds(PROD_OFF, PCARD1), :])


def _emb_kernel(idx_hbm, tab_hbm, out_hbm, idx_v, buf_v, gsem, wsem):
    c = lax.axis_index("c")
    s = lax.axis_index("s")
    wid = s * 2 + c
    b0 = wid * BPW

    def do_batch(g, carry):
        b = b0 + g
        pltpu.sync_copy(idx_hbm.at[pl.ds(b * SLAB, SLAB)], idx_v)

        @pl.when(g > 0)
        def _():
            pltpu.make_async_copy(
                tab_hbm.at[pl.ds(0, SLAB)], buf_v, wsem).wait()

        def gat(j, carry2):
            pltpu.async_copy(
                tab_hbm.at[idx_v.at[pl.ds(j * 128, 128)]],
                buf_v.at[pl.ds(j * 128, 128), :],
                gsem,
            )
            return carry2

        lax.fori_loop(0, NS, gat, 0)
        pltpu.async_copy(
            tab_hbm.at[idx_v.at[pl.ds(NS * 128, TAIL)]],
            buf_v.at[pl.ds(NS * 128, TAIL), :],
            gsem,
        )

        pltpu.make_async_copy(
            tab_hbm.at[pl.ds(0, SLAB)], buf_v, gsem).wait()

        pltpu.async_copy(buf_v, out_hbm.at[pl.ds(b * SLAB, SLAB)], wsem)
        return carry

    lax.fori_loop(0, BPW, do_batch, 0)
    pltpu.make_async_copy(tab_hbm.at[pl.ds(0, SLAB)], buf_v, wsem).wait()


_MESH = dict(core_axis_name="c", subcore_axis_name="s")


@jax.jit
def _run(idx_flat, tables, product_table):
    flatten = functools.partial(
        pl.kernel,
        mesh=plsc.VectorSubcoreMesh(**_MESH),
        compiler_params=pltpu.CompilerParams(use_tc_tiling_on_sc=False),
        out_type=jax.ShapeDtypeStruct((TROWS, EMB), jnp.float32),
        scratch_types=[
            pltpu.VMEM((CHROWS, EMB), jnp.float32),
            pltpu.SemaphoreType.DMA,
        ],
    )(_flat_kernel)
    tab_all = flatten(tables, product_table)

    gather = functools.partial(
        pl.kernel,
        mesh=plsc.VectorSubcoreMesh(**_MESH),
        compiler_params=pltpu.CompilerParams(use_tc_tiling_on_sc=False),
        out_type=jax.ShapeDtypeStruct((M, EMB), jnp.float32),
        scratch_types=[
            pltpu.VMEM((SLAB,), jnp.int32),
            pltpu.VMEM((SLAB, EMB), jnp.float32),
            pltpu.SemaphoreType.DMA,
            pltpu.SemaphoreType.DMA,
        ],
    )(_emb_kernel)
    return gather(idx_flat, tab_all)


def kernel(indices, product, tables, product_table):
    idxf = indices.astype(jnp.int32) + (
        jnp.arange(F, dtype=jnp.int32) * PITCH)[:, None, None]
    pidx = product.astype(jnp.int32) + PROD_OFF
    slab = jnp.concatenate(
        [
            idxf.transpose(1, 2, 0),
            jnp.broadcast_to(pidx[:, None, None], (B, L, 1)),
        ],
        axis=2,
    )
    out_flat = _run(slab.reshape(M), tables, product_table)
    return out_flat.reshape(B, L, NSLOT * EMB)


# speedup vs baseline: 1.5834x; 1.5834x over previous
import functools

import jax
import jax.numpy as jnp
from jax import lax
from jax.experimental import pallas as pl
from jax.experimental.pallas import tpu as pltpu
from jax.experimental.pallas import tpu_sc as plsc

F = 26
B = 1024
L = 200
CARD1 = 100001
PITCH = 100032
PCARD1 = 1001
PROD_OFF = F * PITCH
TROWS = PROD_OFF + 1024
EMB = 16
NSLOT = F + 1
SLAB = L * NSLOT
M = B * SLAB
NW = 32
BPW = B // NW
NS = SLAB // 128
TAIL = SLAB - NS * 128


def _emb_kernel(idx_hbm, tab_hbm, out_hbm, idx_v, buf_v, gsem, wsem):
    c = lax.axis_index("c")
    s = lax.axis_index("s")
    wid = s * 2 + c
    b0 = wid * BPW

    def do_batch(g, carry):
        b = b0 + g
        pltpu.sync_copy(idx_hbm.at[pl.ds(b * SLAB, SLAB)], idx_v)

        @pl.when(g > 0)
        def _():
            pltpu.make_async_copy(
                tab_hbm.at[pl.ds(0, SLAB)], buf_v, wsem).wait()

        def gat(j, carry2):
            pltpu.async_copy(
                tab_hbm.at[idx_v.at[pl.ds(j * 128, 128)]],
                buf_v.at[pl.ds(j * 128, 128), :],
                gsem,
            )
            return carry2

        lax.fori_loop(0, NS, gat, 0)
        pltpu.async_copy(
            tab_hbm.at[idx_v.at[pl.ds(NS * 128, TAIL)]],
            buf_v.at[pl.ds(NS * 128, TAIL), :],
            gsem,
        )

        pltpu.make_async_copy(
            tab_hbm.at[pl.ds(0, SLAB)], buf_v, gsem).wait()

        pltpu.async_copy(buf_v, out_hbm.at[pl.ds(b * SLAB, SLAB)], wsem)
        return carry

    lax.fori_loop(0, BPW, do_batch, 0)
    pltpu.make_async_copy(tab_hbm.at[pl.ds(0, SLAB)], buf_v, wsem).wait()


@jax.jit
def _run(idx_flat, tab_all):
    mesh = plsc.VectorSubcoreMesh(core_axis_name="c", subcore_axis_name="s")
    kfn = functools.partial(
        pl.kernel,
        mesh=mesh,
        compiler_params=pltpu.CompilerParams(use_tc_tiling_on_sc=False),
        out_type=jax.ShapeDtypeStruct((M, EMB), jnp.float32),
        scratch_types=[
            pltpu.VMEM((SLAB,), jnp.int32),
            pltpu.VMEM((SLAB, EMB), jnp.float32),
            pltpu.SemaphoreType.DMA,
            pltpu.SemaphoreType.DMA,
        ],
    )(_emb_kernel)
    return kfn(idx_flat, tab_all)


def kernel(indices, product, tables, product_table):
    zf = jnp.zeros((PITCH - CARD1, EMB), jnp.float32)
    zt = jnp.zeros((TROWS - PROD_OFF - PCARD1, EMB), jnp.float32)
    pieces = []
    for f in range(F):
        pieces.append(tables[f])
        pieces.append(zf)
    pieces.append(product_table)
    pieces.append(zt)
    tab_all = jnp.concatenate(pieces, axis=0)

    idxf = indices.astype(jnp.int32) + (
        jnp.arange(F, dtype=jnp.int32) * PITCH)[:, None, None]
    pidx = product.astype(jnp.int32) + PROD_OFF
    slab = jnp.concatenate(
        [
            idxf.transpose(1, 2, 0),
            jnp.broadcast_to(pidx[:, None, None], (B, L, 1)),
        ],
        axis=2,
    )
    out_flat = _run(slab.reshape(M), tab_all)
    return out_flat.reshape(B, L, NSLOT * EMB)
